# baseline (speedup 1.0000x reference)
import jax
import jax.numpy as jnp
from jax.experimental import pallas as pl
from jax.experimental.pallas import tpu as pltpu

_BLOCK = 2048


def _rq_body(x_ref, pit_ref, pi_ref, pib_ref, cb_ref, out_ref):
    xb = x_ref[:]
    norm2 = jnp.sum(xb * xb, axis=1, keepdims=True)
    rn = jax.lax.rsqrt(norm2)
    rn = rn * (1.5 - (0.5 * norm2) * (rn * rn))
    norm = norm2 * rn
    xr_un = jnp.dot(xb, pit_ref[:], preferred_element_type=jnp.float32)
    xr = xr_un * rn

    k = cb_ref.shape[1]
    c0 = cb_ref[0, 0]
    step = (cb_ref[0, k - 1] - c0) / (k - 1)
    idx = jnp.clip(jnp.round((xr - c0) / step), 0.0, float(k - 1))
    xq = c0 + idx * step
    res = xr - xq
    scale = jnp.mean(jnp.abs(res), axis=1, keepdims=True)
    sgn = jnp.where(res >= 0.0, 1.0, -1.0).astype(jnp.bfloat16)

    out_q = jnp.dot(xq, pi_ref[:], preferred_element_type=jnp.float32)
    out_s = jnp.dot(sgn, pib_ref[:], preferred_element_type=jnp.float32)
    out_ref[:] = (out_q + scale * out_s) * norm


def kernel(x, Pi, centroids):
    n, d = x.shape
    k = centroids.shape[0]
    cb = centroids.reshape(1, k)
    return pl.pallas_call(
        _rq_body,
        grid=(n // _BLOCK,),
        in_specs=[
            pl.BlockSpec((_BLOCK, d), lambda i: (i, 0)),
            pl.BlockSpec((d, d), lambda i: (0, 0)),
            pl.BlockSpec((d, d), lambda i: (0, 0)),
            pl.BlockSpec((d, d), lambda i: (0, 0)),
            pl.BlockSpec((1, k), lambda i: (0, 0)),
        ],
        out_specs=pl.BlockSpec((_BLOCK, d), lambda i: (i, 0)),
        out_shape=jax.ShapeDtypeStruct((n, d), x.dtype),
        compiler_params=pltpu.CompilerParams(
            dimension_semantics=("parallel",),
        ),
    )(x, Pi.T, Pi, Pi.astype(jnp.bfloat16), cb)


# speedup vs baseline: 1.2633x; 1.2633x over previous
import jax
import jax.numpy as jnp
from jax.experimental import pallas as pl
from jax.experimental.pallas import tpu as pltpu

_BLOCK = 8192


def _rq_body(x_ref, pit_ref, pi_ref, cb_ref, out_ref):
    xb = x_ref[:]
    norm = jnp.sqrt(jnp.sum(xb * xb, axis=1, keepdims=True))
    xn = xb / (norm + 1e-8)
    xr = jnp.dot(xn, pit_ref[:], preferred_element_type=jnp.float32)

    k = cb_ref.shape[1]
    c0 = cb_ref[0, 0]
    step = (cb_ref[0, k - 1] - c0) / (k - 1)
    idx = jnp.clip(jnp.round((xr - c0) / step), 0.0, float(k - 1))
    xq = c0 + idx * step
    res = xr - xq
    scale = jnp.mean(jnp.abs(res), axis=1, keepdims=True)
    xc = xq + jnp.where(res >= 0.0, scale, -scale)

    out = jnp.dot(xc, pi_ref[:], preferred_element_type=jnp.float32)
    out_ref[:] = out * norm


def kernel(x, Pi, centroids):
    n, d = x.shape
    k = centroids.shape[0]
    cb = centroids.reshape(1, k)
    return pl.pallas_call(
        _rq_body,
        grid=(n // _BLOCK,),
        in_specs=[
            pl.BlockSpec((_BLOCK, d), lambda i: (i, 0)),
            pl.BlockSpec((d, d), lambda i: (0, 0)),
            pl.BlockSpec((d, d), lambda i: (0, 0)),
            pl.BlockSpec((1, k), lambda i: (0, 0)),
        ],
        out_specs=pl.BlockSpec((_BLOCK, d), lambda i: (i, 0)),
        out_shape=jax.ShapeDtypeStruct((n, d), x.dtype),
        compiler_params=pltpu.CompilerParams(
            dimension_semantics=("parallel",),
        ),
    )(x, Pi.T, Pi, cb)


# speedup vs baseline: 1.3217x; 1.0462x over previous
import jax
import jax.numpy as jnp
from jax.experimental import pallas as pl
from jax.experimental.pallas import tpu as pltpu

_BLOCK = 8192


def _rq_body(x_ref, pi_ref, cb_ref, out_ref):
    xb = x_ref[:]
    norm = jnp.sqrt(jnp.sum(xb * xb, axis=1, keepdims=True))
    xn = xb / (norm + 1e-8)
    xr = jax.lax.dot_general(xn, pi_ref[:], (((1,), (1,)), ((), ())),
                             preferred_element_type=jnp.float32)

    k = cb_ref.shape[1]
    c0 = cb_ref[0, 0]
    step = (cb_ref[0, k - 1] - c0) / (k - 1)
    idx = jnp.clip(jnp.round((xr - c0) / step), 0.0, float(k - 1))
    xq = c0 + idx * step
    res = xr - xq
    scale = jnp.mean(jnp.abs(res), axis=1, keepdims=True)
    xc = xq + jnp.where(res >= 0.0, scale, -scale)

    out = jnp.dot(xc, pi_ref[:], preferred_element_type=jnp.float32)
    out_ref[:] = out * norm


def kernel(x, Pi, centroids):
    n, d = x.shape
    k = centroids.shape[0]
    cb = centroids.reshape(1, k)
    return pl.pallas_call(
        _rq_body,
        grid=(n // _BLOCK,),
        in_specs=[
            pl.BlockSpec((_BLOCK, d), lambda i: (i, 0)),
            pl.BlockSpec((d, d), lambda i: (0, 0)),
            pl.BlockSpec((1, k), lambda i: (0, 0)),
        ],
        out_specs=pl.BlockSpec((_BLOCK, d), lambda i: (i, 0)),
        out_shape=jax.ShapeDtypeStruct((n, d), x.dtype),
        compiler_params=pltpu.CompilerParams(
            dimension_semantics=("parallel",),
        ),
    )(x, Pi, cb)
